# baseline (speedup 1.0000x reference)
import functools

import jax
import jax.numpy as jnp
from jax import lax
from jax.experimental import pallas as pl
from jax.experimental.pallas import tpu as pltpu
from jax.experimental.pallas import tpu_sc as plsc

N, K, H, W = 4, 8, 256, 256
HW = H * W
C = 32
P = 100000
NC, NS, L = 2, 16, 16
NW = NC * NS
PPW = (N * HW) // NW
PARTS = NW // N
SUP = 1024
HROWS = SUP // W
NSUP = PPW // SUP
CH = 128
SUBS = SUP // CH
IB = 128


def _tree_sum(xs):
    while len(xs) > 1:
        xs = [xs[j] + xs[j + 1] for j in range(0, len(xs), 2)]
    return xs[0]


def _sc_composite(table, frag, alpha):
    mesh = plsc.VectorSubcoreMesh(core_axis_name="c", subcore_axis_name="s")

    @functools.partial(
        pl.kernel,
        mesh=mesh,
        compiler_params=pltpu.CompilerParams(use_tc_tiling_on_sc=False,
                                             needs_layout_passes=False),
        out_type=jax.ShapeDtypeStruct((N, C, H, W), jnp.float32),
        scratch_types=[
            pltpu.VMEM((2, K, HROWS, W), jnp.int32),
            pltpu.VMEM((2, K, HROWS, W), jnp.float32),
            pltpu.VMEM((K, SUP), jnp.float32),
            pltpu.VMEM((2, K, CH, C), jnp.bfloat16),
            pltpu.VMEM((C, CH + 1), jnp.float32),
            pltpu.SemaphoreType.DMA,
            pltpu.SemaphoreType.DMA,
        ],
    )
    def k(table_hbm, frag_hbm, alpha_hbm, out_hbm,
          idx_v, alpha_v, w_v, rows_v, acc_v, sem_g, sem_s):
        wid = lax.axis_index("s") * NC + lax.axis_index("c")
        n = wid // PARTS
        base_h = (wid % PARTS) * (PPW // W)

        def frag_slice(h0):
            return frag_hbm.at[n, :, pl.ds(h0, HROWS), :]

        def alpha_slice(h0):
            return alpha_hbm.at[n, :, pl.ds(h0, HROWS), :]

        def sup_h(si):
            return pl.multiple_of(base_h + si * HROWS, HROWS)

        def idx_ref(buf, kk, sj):
            return idx_v.at[buf, kk, sj // 2,
                            pl.ds(pl.multiple_of((sj % 2) * IB, IB), IB)]

        pltpu.sync_copy(frag_slice(sup_h(0)), idx_v.at[0])
        pltpu.sync_copy(alpha_slice(sup_h(0)), alpha_v.at[0])
        for kk in range(K):
            pltpu.async_copy(table_hbm.at[idx_v.at[0, kk, 0, pl.ds(0, IB)]],
                             rows_v.at[0, kk], sem_g)

        def sup_body(si, _):
            b = si % 2
            h0 = sup_h(si)

            @pl.when(si + 1 < NSUP)
            def _():
                pltpu.async_copy(frag_slice(sup_h(si + 1)),
                                 idx_v.at[1 - b], sem_s)
                pltpu.async_copy(alpha_slice(sup_h(si + 1)),
                                 alpha_v.at[1 - b], sem_s)

            def wgrp(g, _):
                jj = g // (W // L)
                off = (g % (W // L)) * L
                t = jnp.ones((L,), jnp.float32)
                for kk in range(K):
                    a = alpha_v[b, kk, jj, pl.ds(off, L)]
                    w_v[kk, pl.ds(g * L, L)] = a * t
                    t = t * (1.0 - a)
                return 0
            lax.fori_loop(0, SUP // L, wgrp, 0)

            def sub_body(sj, _):
                rp = sj % 2
                np_ = (sj + 1) % 2

                for kk in range(K):
                    pltpu.make_async_copy(
                        table_hbm.at[idx_ref(b, kk, sj)],
                        rows_v.at[rp, kk], sem_g).wait()

                @pl.when(sj < SUBS - 1)
                def _():
                    for kk in range(K):
                        pltpu.async_copy(
                            table_hbm.at[idx_ref(b, kk, sj + 1)],
                            rows_v.at[np_, kk], sem_g)

                @pl.when(jnp.logical_and(sj == SUBS - 1, si < NSUP - 1))
                def _():
                    pltpu.make_async_copy(frag_slice(sup_h(si + 1)),
                                          idx_v.at[1 - b], sem_s).wait()
                    pltpu.make_async_copy(alpha_slice(sup_h(si + 1)),
                                          alpha_v.at[1 - b], sem_s).wait()
                    for kk in range(K):
                        pltpu.async_copy(
                            table_hbm.at[idx_v.at[1 - b, kk, 0,
                                                  pl.ds(0, IB)]],
                            rows_v.at[np_, kk], sem_g)

                rows_ev = 2 * lax.iota(jnp.int32, L)
                rows_od = rows_ev + 1

                def px_body(g, _):
                    p0 = g * L
                    wvs = [w_v[kk, pl.ds(sj * CH + p0, L)] for kk in range(K)]
                    for i in range(L):
                        p = p0 + i
                        ev, od = [], []
                        for kk in range(K):
                            ri = plsc.bitcast(
                                rows_v[rp, kk, p, pl.ds(0, C)], jnp.int32)
                            wk = wvs[kk][i]
                            ev.append(wk * plsc.bitcast(ri << 16,
                                                        jnp.float32))
                            od.append(wk * plsc.bitcast(ri & -65536,
                                                        jnp.float32))
                        col = jnp.full((L,), p, jnp.int32)
                        plsc.store_scatter(acc_v, [rows_ev, col],
                                           _tree_sum(ev))
                        plsc.store_scatter(acc_v, [rows_od, col],
                                           _tree_sum(od))
                    return 0
                lax.fori_loop(0, CH // L, px_body, 0)

                hw = h0 * W + sj * CH
                h_row = hw // W
                w0_ = pl.multiple_of(hw % W, CH)
                pltpu.sync_copy(acc_v.at[:, pl.ds(0, CH)],
                                out_hbm.at[n, :, h_row, pl.ds(w0_, CH)])
                return 0

            lax.fori_loop(0, SUBS, sub_body, 0)
            return 0

        lax.fori_loop(0, NSUP, sup_body, 0)

    return k(table, frag, alpha)


def kernel(fragments, alphas, ptclds):
    frag = fragments.astype(jnp.int32)
    table = ptclds.T.astype(jnp.bfloat16)
    return _sc_composite(table, frag, alphas)


# speedup vs baseline: 1.1711x; 1.1711x over previous
import functools

import jax
import jax.numpy as jnp
from jax import lax
from jax.experimental import pallas as pl
from jax.experimental.pallas import tpu as pltpu
from jax.experimental.pallas import tpu_sc as plsc

N, K, H, W = 4, 8, 256, 256
HW = H * W
C = 32
P = 100000
NC, NS, L = 2, 16, 16
NW = NC * NS
PPW = (N * HW) // NW
PARTS = NW // N
SUP = 1024
HROWS = SUP // W
NSUP = PPW // SUP
CH = 128
SUBS = SUP // CH
IB = 128
TR, TC_ = 8, 128
HB, WB = H // TR, W // TC_


def _tree_sum(xs):
    while len(xs) > 1:
        xs = [xs[j] + xs[j + 1] for j in range(0, len(xs), 2)]
    return xs[0]


def _sc_composite(table, frag, alpha):
    mesh = plsc.VectorSubcoreMesh(core_axis_name="c", subcore_axis_name="s")

    @functools.partial(
        pl.kernel,
        mesh=mesh,
        compiler_params=pltpu.CompilerParams(use_tc_tiling_on_sc=False,
                                             needs_layout_passes=False),
        out_type=jax.ShapeDtypeStruct((N, C, HB, WB, TR, TC_), jnp.float32),
        scratch_types=[
            pltpu.VMEM((2, K, WB, HROWS, IB), jnp.int32),
            pltpu.VMEM((2, K, WB, HROWS, TC_), jnp.float32),
            pltpu.VMEM((K, WB, HROWS, TC_), jnp.float32),
            pltpu.VMEM((2, K, CH, C), jnp.bfloat16),
            pltpu.VMEM((C, CH + 1), jnp.float32),
            pltpu.SemaphoreType.DMA,
            pltpu.SemaphoreType.DMA,
        ],
    )
    def k(table_hbm, frag_hbm, alpha_hbm, out_hbm,
          idx_v, alpha_v, w_v, rows_v, acc_v, sem_g, sem_s):
        wid = lax.axis_index("s") * NC + lax.axis_index("c")
        n = wid // PARTS
        base_h = (wid % PARTS) * (PPW // W)

        def hb_r0(si):
            h_abs = base_h + si * HROWS
            return h_abs // TR, pl.multiple_of(h_abs % TR, HROWS)

        def frag_slice(si):
            hb, r0 = hb_r0(si)
            return frag_hbm.at[n, :, hb, :, pl.ds(r0, HROWS), :]

        def alpha_slice(si):
            hb, r0 = hb_r0(si)
            return alpha_hbm.at[n, :, hb, :, pl.ds(r0, HROWS), :]

        def idx_ref(buf, kk, sj):
            return idx_v.at[buf, kk, sj % WB, sj // WB]

        pltpu.sync_copy(frag_slice(0), idx_v.at[0])
        pltpu.sync_copy(alpha_slice(0), alpha_v.at[0])
        for kk in range(K):
            pltpu.async_copy(table_hbm.at[idx_v.at[0, kk, 0, 0]],
                             rows_v.at[0, kk], sem_g)

        def sup_body(si, _):
            b = si % 2

            @pl.when(si + 1 < NSUP)
            def _():
                pltpu.async_copy(frag_slice(si + 1), idx_v.at[1 - b], sem_s)
                pltpu.async_copy(alpha_slice(si + 1), alpha_v.at[1 - b],
                                 sem_s)

            def wgrp(g, _):
                wb = g // (SUP // L // WB)
                rr = (g // (TC_ // L)) % HROWS
                off = (g % (TC_ // L)) * L
                t = jnp.ones((L,), jnp.float32)
                for kk in range(K):
                    a = alpha_v[b, kk, wb, rr, pl.ds(off, L)]
                    w_v[kk, wb, rr, pl.ds(off, L)] = a * t
                    t = t * (1.0 - a)
                return 0
            lax.fori_loop(0, SUP // L, wgrp, 0)

            def sub_body(sj, _):
                rp = sj % 2
                np_ = (sj + 1) % 2
                wb = sj % WB
                rr = sj // WB

                for kk in range(K):
                    pltpu.make_async_copy(
                        table_hbm.at[idx_ref(b, kk, sj)],
                        rows_v.at[rp, kk], sem_g).wait()

                @pl.when(sj < SUBS - 1)
                def _():
                    for kk in range(K):
                        pltpu.async_copy(
                            table_hbm.at[idx_ref(b, kk, sj + 1)],
                            rows_v.at[np_, kk], sem_g)

                @pl.when(jnp.logical_and(sj == SUBS - 1, si < NSUP - 1))
                def _():
                    pltpu.make_async_copy(frag_slice(si + 1),
                                          idx_v.at[1 - b], sem_s).wait()
                    pltpu.make_async_copy(alpha_slice(si + 1),
                                          alpha_v.at[1 - b], sem_s).wait()
                    for kk in range(K):
                        pltpu.async_copy(
                            table_hbm.at[idx_v.at[1 - b, kk, 0, 0]],
                            rows_v.at[np_, kk], sem_g)

                rows_ev = 2 * lax.iota(jnp.int32, L)
                rows_od = rows_ev + 1

                def px_body(g, _):
                    p0 = g * L
                    wvs = [w_v[kk, wb, rr, pl.ds(p0, L)] for kk in range(K)]
                    for i in range(L):
                        p = p0 + i
                        ev, od = [], []
                        for kk in range(K):
                            ri = plsc.bitcast(
                                rows_v[rp, kk, p, pl.ds(0, C)], jnp.int32)
                            wk = wvs[kk][i]
                            ev.append(wk * plsc.bitcast(ri << 16,
                                                        jnp.float32))
                            od.append(wk * plsc.bitcast(ri & -65536,
                                                        jnp.float32))
                        col = jnp.full((L,), p, jnp.int32)
                        plsc.store_scatter(acc_v, [rows_ev, col],
                                           _tree_sum(ev))
                        plsc.store_scatter(acc_v, [rows_od, col],
                                           _tree_sum(od))
                    return 0
                lax.fori_loop(0, CH // L, px_body, 0)

                hb, r0 = hb_r0(si)
                pltpu.sync_copy(acc_v.at[:, pl.ds(0, CH)],
                                out_hbm.at[n, :, hb, wb, r0 + rr, :])
                return 0

            lax.fori_loop(0, SUBS, sub_body, 0)
            return 0

        lax.fori_loop(0, NSUP, sup_body, 0)

    return k(table, frag, alpha)


def kernel(fragments, alphas, ptclds):
    frag = (fragments.astype(jnp.int32)
            .reshape(N, K, HB, TR, WB, TC_).transpose(0, 1, 2, 4, 3, 5))
    alpha = (alphas.reshape(N, K, HB, TR, WB, TC_)
             .transpose(0, 1, 2, 4, 3, 5))
    table = ptclds.T.astype(jnp.bfloat16)
    out6 = _sc_composite(table, frag, alpha)
    return (out6.transpose(0, 1, 2, 4, 3, 5)
            .reshape(N, C, H, W))
